# Initial kernel scaffold; baseline (speedup 1.0000x reference)
import jax
import jax.numpy as jnp
from jax.experimental import pallas as pl


def kernel(x, W0a, W0b, Wh0, bh0, Wh1, bh1, Wout, bout, edge_index):
    raise NotImplementedError("write your pallas kernel here")


# speedup vs baseline: 3.5198x; 3.5198x over previous
import functools

import jax
import jax.numpy as jnp
from jax import lax
from jax.experimental import pallas as pl
from jax.experimental.pallas import tpu as pltpu
from jax.experimental.pallas import tpu_sc as plsc

NS = 16
NC = 2
CHUNK = 128
GPAD = 16


@functools.lru_cache(maxsize=None)
def _make_agg(n, dh, e_pad):
    edges_per_tile = e_pad // NS
    chunks = edges_per_tile // CHUNK
    assert edges_per_tile % CHUNK == 0
    rows_per_tile = (n // NS) & ~7
    rows_rem = n - NS * rows_per_tile
    assert rows_rem % 8 == 0
    n_acc = n + GPAD
    mesh = plsc.VectorSubcoreMesh(core_axis_name="c", subcore_axis_name="s")

    @functools.partial(
        pl.kernel,
        mesh=mesh,
        out_type=[
            jax.ShapeDtypeStruct((n, dh), jnp.float32),
            jax.ShapeDtypeStruct((n, dh), jnp.float32),
        ],
        scratch_types=[
            pltpu.VMEM((CHUNK,), jnp.int32),
            pltpu.VMEM((CHUNK,), jnp.int32),
            pltpu.VMEM((CHUNK, dh), jnp.float32),
            pltpu.VMEM_SHARED((n_acc, dh), jnp.float32),
            pltpu.SemaphoreType.DMA,
        ],
    )
    def agg(h0, h1, rowi, coli, out0, out1, col_v, row_v, rows_v, acc, sem):
        c = lax.axis_index("c")
        s = lax.axis_index("s")

        def work(h_ref, out_ref):
            nb = s * rows_per_tile
            pltpu.sync_copy(h_ref.at[pl.ds(nb, rows_per_tile)],
                            acc.at[pl.ds(nb, rows_per_tile)])
            if rows_rem:
                @pl.when(s == 0)
                def _():
                    pltpu.sync_copy(h_ref.at[pl.ds(NS * rows_per_tile, rows_rem)],
                                    acc.at[pl.ds(NS * rows_per_tile, rows_rem)])
            plsc.subcore_barrier()

            def chunk_body(i, carry):
                base = s * edges_per_tile + i * CHUNK
                pltpu.sync_copy(coli.at[pl.ds(base, CHUNK)], col_v)
                pltpu.sync_copy(rowi.at[pl.ds(base, CHUNK)], row_v)
                for j in range(CHUNK // 16):
                    sl = pl.ds(j * 16, 16)
                    r = row_v[sl]
                    cc = col_v[sl]
                    row_v[sl] = jnp.where(r == cc, n, r)
                pltpu.async_copy(h_ref.at[col_v], rows_v, sem).wait()
                pltpu.sync_copy(rows_v, acc.at[row_v], add=True)
                return carry

            lax.fori_loop(0, chunks, chunk_body, 0)
            plsc.subcore_barrier()
            pltpu.sync_copy(acc.at[pl.ds(nb, rows_per_tile)],
                            out_ref.at[pl.ds(nb, rows_per_tile)])
            if rows_rem:
                @pl.when(s == 0)
                def _():
                    pltpu.sync_copy(acc.at[pl.ds(NS * rows_per_tile, rows_rem)],
                                    out_ref.at[pl.ds(NS * rows_per_tile, rows_rem)])

        @pl.when(c == 0)
        def _():
            work(h0, out0)

        @pl.when(c == 1)
        def _():
            work(h1, out1)

    return agg


@functools.lru_cache(maxsize=None)
def _make_agg_in(n, d, e_pad):
    edges_per_sc = e_pad // NC
    edges_per_tile = edges_per_sc // NS
    chunks = edges_per_tile // CHUNK
    assert edges_per_tile % CHUNK == 0
    rows_per_tile = (n // NS) & ~7
    rows_rem = n - NS * rows_per_tile
    assert rows_rem % 8 == 0
    n_acc = n + GPAD
    mesh = plsc.VectorSubcoreMesh(core_axis_name="c", subcore_axis_name="s")

    @functools.partial(
        pl.kernel,
        mesh=mesh,
        out_type=[
            jax.ShapeDtypeStruct((n, d), jnp.float32),
            jax.ShapeDtypeStruct((n, d), jnp.float32),
        ],
        scratch_types=[
            pltpu.VMEM((CHUNK,), jnp.int32),
            pltpu.VMEM((CHUNK,), jnp.int32),
            pltpu.VMEM((CHUNK, d), jnp.float32),
            pltpu.VMEM_SHARED((n_acc, d), jnp.float32),
            pltpu.SemaphoreType.DMA,
        ],
    )
    def agg(x, rowi, coli, zbuf, out0, out1, col_v, row_v, rows_v, acc, sem):
        c = lax.axis_index("c")
        s = lax.axis_index("s")
        nb = s * rows_per_tile

        @pl.when(c == 0)
        def _():
            pltpu.sync_copy(x.at[pl.ds(nb, rows_per_tile)],
                            acc.at[pl.ds(nb, rows_per_tile)])
            if rows_rem:
                @pl.when(s == 0)
                def _():
                    pltpu.sync_copy(x.at[pl.ds(NS * rows_per_tile, rows_rem)],
                                    acc.at[pl.ds(NS * rows_per_tile, rows_rem)])

        @pl.when(c == 1)
        def _():
            pltpu.sync_copy(zbuf, acc.at[pl.ds(nb, rows_per_tile)])
            if rows_rem:
                @pl.when(s == 0)
                def _():
                    pltpu.sync_copy(zbuf.at[pl.ds(0, rows_rem)],
                                    acc.at[pl.ds(NS * rows_per_tile, rows_rem)])

        plsc.subcore_barrier()

        def chunk_body(i, carry):
            base = c * edges_per_sc + s * edges_per_tile + i * CHUNK
            pltpu.sync_copy(coli.at[pl.ds(base, CHUNK)], col_v)
            pltpu.sync_copy(rowi.at[pl.ds(base, CHUNK)], row_v)
            for j in range(CHUNK // 16):
                sl = pl.ds(j * 16, 16)
                r = row_v[sl]
                cc = col_v[sl]
                row_v[sl] = jnp.where(r == cc, n, r)
            pltpu.async_copy(x.at[col_v], rows_v, sem).wait()
            pltpu.sync_copy(rows_v, acc.at[row_v], add=True)
            return carry

        lax.fori_loop(0, chunks, chunk_body, 0)
        plsc.subcore_barrier()

        def writeout(out_ref):
            pltpu.sync_copy(acc.at[pl.ds(nb, rows_per_tile)],
                            out_ref.at[pl.ds(nb, rows_per_tile)])
            if rows_rem:
                @pl.when(s == 0)
                def _():
                    pltpu.sync_copy(acc.at[pl.ds(NS * rows_per_tile, rows_rem)],
                                    out_ref.at[pl.ds(NS * rows_per_tile, rows_rem)])

        @pl.when(c == 0)
        def _():
            writeout(out0)

        @pl.when(c == 1)
        def _():
            writeout(out1)

    return agg


@functools.lru_cache(maxsize=None)
def _make_mlp(n, blk, din, dout, mode):
    dhi, dho = din // 2, dout // 2
    grid = (n // blk,)
    a_spec = [
        pl.BlockSpec((blk, dhi), lambda i: (i, 0)),
        pl.BlockSpec((blk, dhi), lambda i: (i, 0)),
    ]

    if mode == "in":
        def body(a0, a1, w1, w2, o0, o1):
            a = a0[...] + a1[...]
            t = jnp.maximum(jnp.dot(a, w1[...],
                                    preferred_element_type=jnp.float32), 0.0)
            z = jnp.maximum(jnp.dot(t, w2[...],
                                    preferred_element_type=jnp.float32), 0.0)
            o0[...] = z[:, :dho]
            o1[...] = z[:, dho:]

        in_specs = [
            pl.BlockSpec((blk, din), lambda i: (i, 0)),
            pl.BlockSpec((blk, din), lambda i: (i, 0)),
        ] + [
            pl.BlockSpec((din, dout), lambda i: (0, 0)),
            pl.BlockSpec((dout, dout), lambda i: (0, 0)),
        ]
        out_shape = [jax.ShapeDtypeStruct((n, dho), jnp.float32)] * 2
        out_specs = [pl.BlockSpec((blk, dho), lambda i: (i, 0))] * 2
    else:
        trailing_relu = mode == "hidden"

        def body(a0, a1, w1, b1, *outs):
            a = jnp.concatenate([a0[...], a1[...]], axis=1)
            a = jnp.maximum(a, 0.0)
            z = jnp.dot(a, w1[...], preferred_element_type=jnp.float32) + b1[...]
            if trailing_relu:
                z = jnp.maximum(z, 0.0)
            if len(outs) == 2:
                outs[0][...] = z[:, :dho]
                outs[1][...] = z[:, dho:]
            else:
                outs[0][...] = z

        in_specs = a_spec + [
            pl.BlockSpec((din, dout), lambda i: (0, 0)),
            pl.BlockSpec((1, dout), lambda i: (0, 0)),
        ]
        if mode == "hidden":
            out_shape = [jax.ShapeDtypeStruct((n, dho), jnp.float32)] * 2
            out_specs = [pl.BlockSpec((blk, dho), lambda i: (i, 0))] * 2
        else:
            out_shape = jax.ShapeDtypeStruct((n, dout), jnp.float32)
            out_specs = pl.BlockSpec((blk, dout), lambda i: (i, 0))

    return pl.pallas_call(
        body,
        grid=grid,
        in_specs=in_specs,
        out_specs=out_specs,
        out_shape=out_shape,
    )


def kernel(x, W0a, W0b, Wh0, bh0, Wh1, bh1, Wout, bout, edge_index):
    n, d_in = x.shape
    d = W0a.shape[1]
    e = edge_index.shape[1]

    stride = NC * NS * CHUNK
    e_pad = ((e + stride - 1) // stride) * stride
    row = edge_index[0]
    col = edge_index[1]
    if e_pad != e:
        pad = jnp.zeros((e_pad - e,), jnp.int32)
        row = jnp.concatenate([row, pad])
        col = jnp.concatenate([col, pad])

    hd = d // 2
    rows_per_tile = (n // NS) & ~7
    zbuf = jnp.zeros((rows_per_tile, d_in), jnp.float32)

    agg_in = _make_agg_in(n, d_in, e_pad)
    agg_hid = _make_agg(n, hd, e_pad)
    blk = 1000
    mlp_in = _make_mlp(n, blk, d_in, d, "in")
    mlp_hid = _make_mlp(n, blk, d, d, "hidden")
    mlp_out = _make_mlp(n, blk, d, d, "out")

    a0, a1 = agg_in(x, row, col, zbuf)
    h0, h1 = mlp_in(a0, a1, W0a, W0b)
    a0, a1 = agg_hid(h0, h1, row, col)
    h0, h1 = mlp_hid(a0, a1, Wh0, bh0.reshape(1, -1))
    a0, a1 = agg_hid(h0, h1, row, col)
    h0, h1 = mlp_hid(a0, a1, Wh1, bh1.reshape(1, -1))
    a0, a1 = agg_hid(h0, h1, row, col)
    return mlp_out(a0, a1, Wout, bout.reshape(1, -1))
